# Initial kernel scaffold; baseline (speedup 1.0000x reference)
import jax
import jax.numpy as jnp
from jax.experimental import pallas as pl


def kernel(value_table, pos_table, W, b, inputs):
    raise NotImplementedError("write your pallas kernel here")


# speedup vs baseline: 3.0055x; 3.0055x over previous
import functools

import jax
import jax.numpy as jnp
from jax import lax
from jax.experimental import pallas as pl
from jax.experimental.pallas import tpu as pltpu
from jax.experimental.pallas import tpu_sc as plsc

WORD_VOCAB = 100000
D = 64
NT = 4096 * 200
NC, NS = 2, 16
NW = NC * NS
TOK_PER_W = NT // NW
C = 128
NCHUNK = TOK_PER_W // C


_R = 1000


def _transform_body(vt_ref, pt_ref, w_ref, b_ref, vtt_ref, ptt_ref):
    w = w_ref[...]
    wv = w[:, 0:64]
    wp = w[:, 64:128]
    dn = (((1,), (1,)), ((), ()))
    vtt_ref[...] = (
        lax.dot_general(vt_ref[...], wv, dn, precision=lax.Precision.HIGHEST)
        + b_ref[...]
    )
    ptt_ref[...] = lax.dot_general(
        pt_ref[...], wp, dn, precision=lax.Precision.HIGHEST
    )


def _transform_tables(value_table, pos_table, W, b):
    b2 = b.reshape(1, D)
    return pl.pallas_call(
        _transform_body,
        grid=(WORD_VOCAB // _R,),
        in_specs=[
            pl.BlockSpec((_R, D), lambda i: (i, 0)),
            pl.BlockSpec((_R, D), lambda i: (i, 0)),
            pl.BlockSpec((D, 2 * D), lambda i: (0, 0)),
            pl.BlockSpec((1, D), lambda i: (0, 0)),
        ],
        out_specs=[
            pl.BlockSpec((_R, D), lambda i: (i, 0)),
            pl.BlockSpec((_R, D), lambda i: (i, 0)),
        ],
        out_shape=[
            jax.ShapeDtypeStruct((WORD_VOCAB, D), jnp.float32),
            jax.ShapeDtypeStruct((WORD_VOCAB, D), jnp.float32),
        ],
    )(value_table, pos_table, W, b2)


def _sc_body(vtt, ptt, pt, vi, pi, merged, pos, vi_v, pi_v, va, vb, vc,
             s0, s1, s2):
    wid = lax.axis_index("s") * NC + lax.axis_index("c")

    def chunk(g, carry):
        base = wid * TOK_PER_W + g * C
        pltpu.sync_copy(vi.at[pl.ds(base, C)], vi_v)
        pltpu.sync_copy(pi.at[pl.ds(base, C)], pi_v)
        cp_a = pltpu.async_copy(vtt.at[vi_v], va, s0)
        cp_b = pltpu.async_copy(ptt.at[pi_v], vb, s1)
        cp_c = pltpu.async_copy(pt.at[pi_v], vc, s2)
        cp_a.wait()
        cp_b.wait()

        def tok(t, c2):
            for dd in range(D // 16):
                sl = pl.ds(dd * 16, 16)
                va[t, sl] = jnp.maximum(va[t, sl] + vb[t, sl], 0.0)
            return c2

        lax.fori_loop(0, C, tok, 0)
        cp_c.wait()
        pltpu.sync_copy(va, merged.at[pl.ds(base, C)])
        pltpu.sync_copy(vc, pos.at[pl.ds(base, C)])
        return carry

    lax.fori_loop(0, NCHUNK, chunk, 0)


_sc_gather = functools.partial(
    pl.kernel,
    out_type=[
        jax.ShapeDtypeStruct((NT, D), jnp.float32),
        jax.ShapeDtypeStruct((NT, D), jnp.float32),
    ],
    mesh=plsc.VectorSubcoreMesh(core_axis_name="c", subcore_axis_name="s"),
    compiler_params=pltpu.CompilerParams(use_tc_tiling_on_sc=False),
    scratch_types=[
        pltpu.VMEM((C,), jnp.int32),
        pltpu.VMEM((C,), jnp.int32),
        pltpu.VMEM((C, D), jnp.float32),
        pltpu.VMEM((C, D), jnp.float32),
        pltpu.VMEM((C, D), jnp.float32),
        pltpu.SemaphoreType.DMA,
        pltpu.SemaphoreType.DMA,
        pltpu.SemaphoreType.DMA,
    ],
)(_sc_body)


def kernel(value_table, pos_table, W, b, inputs):
    vtt, ptt = _transform_tables(value_table, pos_table, W, b)
    flat = inputs.reshape(NT, 2)
    vi = flat[:, 0]
    pi = flat[:, 1]
    merged, pos = _sc_gather(vtt, ptt, pos_table, vi, pi)
    return (merged.reshape(4096, 200, D), pos.reshape(4096, 200, D))


# speedup vs baseline: 3.7505x; 1.2478x over previous
import functools

import jax
import jax.numpy as jnp
from jax import lax
from jax.experimental import pallas as pl
from jax.experimental.pallas import tpu as pltpu
from jax.experimental.pallas import tpu_sc as plsc

WORD_VOCAB = 100000
D = 64
B = 4096
L = 200
NC, NS = 2, 16
NW = NC * NS
ROWS_PER_W = B // NW
IBLK = 16
H0, H1 = 104, 96


_R = 1000


def _transform_body(vt_ref, pt_ref, w_ref, b_ref, vtt_ref, ptt_ref):
    w = w_ref[...]
    wv = w[:, 0:64]
    wp = w[:, 64:128]
    dn = (((1,), (1,)), ((), ()))
    vtt_ref[...] = (
        lax.dot_general(vt_ref[...], wv, dn, precision=lax.Precision.HIGHEST)
        + b_ref[...]
    )
    ptt_ref[...] = lax.dot_general(
        pt_ref[...], wp, dn, precision=lax.Precision.HIGHEST
    )


def _transform_tables(value_table, pos_table, W, b):
    b2 = b.reshape(1, D)
    return pl.pallas_call(
        _transform_body,
        grid=(WORD_VOCAB // _R,),
        in_specs=[
            pl.BlockSpec((_R, D), lambda i: (i, 0)),
            pl.BlockSpec((_R, D), lambda i: (i, 0)),
            pl.BlockSpec((D, 2 * D), lambda i: (0, 0)),
            pl.BlockSpec((1, D), lambda i: (0, 0)),
        ],
        out_specs=[
            pl.BlockSpec((_R, D), lambda i: (i, 0)),
            pl.BlockSpec((_R, D), lambda i: (i, 0)),
        ],
        out_shape=[
            jax.ShapeDtypeStruct((WORD_VOCAB, D), jnp.float32),
            jax.ShapeDtypeStruct((WORD_VOCAB, D), jnp.float32),
        ],
    )(value_table, pos_table, W, b2)


def _sc_body(vtt, ptt, pt, vi, pi, merged, pos,
             ivi, ipi, va0, vb0, vc0, va1, vb1, vc1,
             gs0, gs1, ss0, ss1):
    wid = lax.axis_index("s") * NC + lax.axis_index("c")
    base_row = wid * ROWS_PER_W

    def stage_idx(g):
        r0 = base_row + g
        pltpu.sync_copy(vi.at[pl.ds(r0, IBLK), :], ivi)
        pltpu.sync_copy(pi.at[pl.ds(r0, IBLK), :], ipi)

    def gather_copies(g, va, vb, vc, sem):
        jj = lax.rem(g, IBLK)
        out = []
        for off, width in ((0, H0), (H0, H1)):
            sl = pl.ds(off, width)
            dst = pl.ds(off, width)
            out.append(pltpu.make_async_copy(
                vtt.at[ivi.at[jj, sl]], va.at[dst], sem))
            out.append(pltpu.make_async_copy(
                ptt.at[ipi.at[jj, sl]], vb.at[dst], sem))
            out.append(pltpu.make_async_copy(
                pt.at[ipi.at[jj, sl]], vc.at[dst], sem))
        return out

    def issue_gathers(g, va, vb, vc, sem):
        for c in gather_copies(g, va, vb, vc, sem):
            c.start()

    def wait_gathers(g, va, vb, vc, sem):
        for c in gather_copies(g, va, vb, vc, sem):
            c.wait()

    def scatter_copies(g, va, vc, sem):
        row = base_row + g
        return [
            pltpu.make_async_copy(va, merged.at[row], sem),
            pltpu.make_async_copy(vc, pos.at[row], sem),
        ]

    def compute(va, vb):
        def tok(t, carry):
            for dd in range(D // 16):
                sl = pl.ds(dd * 16, 16)
                va[t, sl] = jnp.maximum(va[t, sl] + vb[t, sl], 0.0)
            return carry
        lax.fori_loop(0, L, tok, 0)

    def loop_body(i, carry):
        g0 = 2 * i
        g1 = 2 * i + 1
        @pl.when(lax.rem(g0 + 1, IBLK) == 0)
        def _():
            stage_idx(g0 + 1)
        @pl.when(i >= 1)
        def _():
            for c in scatter_copies(g0 - 1, va1, vc1, ss1):
                c.wait()
        issue_gathers(g0 + 1, va1, vb1, vc1, gs1)
        wait_gathers(g0, va0, vb0, vc0, gs0)
        compute(va0, vb0)
        for c in scatter_copies(g0, va0, vc0, ss0):
            c.start()
        @pl.when(i < (ROWS_PER_W // 2) - 1)
        def _():
            @pl.when(lax.rem(g1 + 1, IBLK) == 0)
            def _():
                stage_idx(g1 + 1)
            for c in scatter_copies(g0, va0, vc0, ss0):
                c.wait()
            issue_gathers(g1 + 1, va0, vb0, vc0, gs0)
        wait_gathers(g1, va1, vb1, vc1, gs1)
        compute(va1, vb1)
        for c in scatter_copies(g1, va1, vc1, ss1):
            c.start()
        return carry

    stage_idx(0)
    issue_gathers(0, va0, vb0, vc0, gs0)
    lax.fori_loop(0, ROWS_PER_W // 2, loop_body, 0)
    for c in scatter_copies(ROWS_PER_W - 2, va0, vc0, ss0):
        c.wait()
    for c in scatter_copies(ROWS_PER_W - 1, va1, vc1, ss1):
        c.wait()


_sc_gather = functools.partial(
    pl.kernel,
    out_type=[
        jax.ShapeDtypeStruct((B, L, D), jnp.float32),
        jax.ShapeDtypeStruct((B, L, D), jnp.float32),
    ],
    mesh=plsc.VectorSubcoreMesh(core_axis_name="c", subcore_axis_name="s"),
    compiler_params=pltpu.CompilerParams(use_tc_tiling_on_sc=False),
    scratch_types=[
        pltpu.VMEM((IBLK, L), jnp.int32),
        pltpu.VMEM((IBLK, L), jnp.int32),
        pltpu.VMEM((L, D), jnp.float32),
        pltpu.VMEM((L, D), jnp.float32),
        pltpu.VMEM((L, D), jnp.float32),
        pltpu.VMEM((L, D), jnp.float32),
        pltpu.VMEM((L, D), jnp.float32),
        pltpu.VMEM((L, D), jnp.float32),
        pltpu.SemaphoreType.DMA,
        pltpu.SemaphoreType.DMA,
        pltpu.SemaphoreType.DMA,
        pltpu.SemaphoreType.DMA,
    ],
)(_sc_body)


def kernel(value_table, pos_table, W, b, inputs):
    vtt, ptt = _transform_tables(value_table, pos_table, W, b)
    vi = inputs[:, :, 0]
    pi = inputs[:, :, 1]
    merged, pos = _sc_gather(vtt, ptt, pos_table, vi, pi)
    return (merged, pos)
